# baseline (speedup 1.0000x reference)
import functools

import jax
import jax.numpy as jnp
from jax import lax
from jax.experimental import pallas as pl
from jax.experimental.pallas import tpu as pltpu
from jax.experimental.pallas import tpu_sc as plsc

N_NODES = 10000
N_EDGES = 320000
DIM = 128

NUM_CORES = 2
NUM_SUBCORES = 16
NUM_TILES = NUM_CORES * NUM_SUBCORES

EDGES_PER_TILE = N_EDGES // NUM_TILES
CHUNK = 128
PHASES = 2
PHASE_STEPS = 40
STEPS = PHASES * PHASE_STEPS
PAD_EDGES_PER_TILE = STEPS * CHUNK
Z_ROWS = N_NODES + 8
ROWS_PER_TILE = 624
ROWS_TAIL = Z_ROWS - NUM_SUBCORES * ROWS_PER_TILE

_ROW_BLK = 2000


def _mlp1_body(x_ref, w1_ref, b1_ref, w2_ref, b2_ref, o_ref):
    h = jnp.maximum(
        jnp.dot(x_ref[...], w1_ref[...], preferred_element_type=jnp.float32)
        + b1_ref[...], 0.0)
    o_ref[...] = jnp.maximum(
        jnp.dot(h, w2_ref[...], preferred_element_type=jnp.float32)
        + b2_ref[...], 0.0)


def _mlp2_body(z0_ref, z1_ref, w3_ref, b3_ref, w4_ref, b4_ref, o_ref):
    z = z0_ref[...] + z1_ref[...]
    h = jnp.maximum(
        jnp.dot(z, w3_ref[...], preferred_element_type=jnp.float32)
        + b3_ref[...], 0.0)
    o_ref[...] = jnp.maximum(
        jnp.dot(h, w4_ref[...], preferred_element_type=jnp.float32)
        + b4_ref[...], 0.0)


_full = pl.BlockSpec((DIM, DIM), lambda i: (0, 0))
_bias = pl.BlockSpec((1, DIM), lambda i: (0, 0))
_rows = pl.BlockSpec((_ROW_BLK, DIM), lambda i: (i, 0))

_mlp1 = pl.pallas_call(
    _mlp1_body,
    grid=(N_NODES // _ROW_BLK,),
    in_specs=[_rows, _full, _bias, _full, _bias],
    out_specs=_rows,
    out_shape=jax.ShapeDtypeStruct((N_NODES, DIM), jnp.float32),
)

_mlp2 = pl.pallas_call(
    _mlp2_body,
    grid=(N_NODES // _ROW_BLK,),
    in_specs=[_rows, _rows, _full, _bias, _full, _bias],
    out_specs=_rows,
    out_shape=jax.ShapeDtypeStruct((N_NODES, DIM), jnp.float32),
)


@functools.partial(
    pl.kernel,
    out_type=jax.ShapeDtypeStruct((NUM_CORES, Z_ROWS, DIM), jnp.float32),
    mesh=plsc.VectorSubcoreMesh(core_axis_name="c", subcore_axis_name="s"),
    scratch_types=[
        pltpu.VMEM((PHASE_STEPS, CHUNK), jnp.int32),
        pltpu.VMEM((PHASE_STEPS, CHUNK), jnp.int32),
        pltpu.VMEM((CHUNK, DIM), jnp.float32),
        pltpu.VMEM((CHUNK, DIM), jnp.float32),
        pltpu.VMEM_SHARED((Z_ROWS, DIM), jnp.float32),
        pltpu.SemaphoreType.DMA,
        pltpu.SemaphoreType.DMA,
    ],
)
def _aggregate(m_hbm, src_hbm, dst_hbm, zeros_hbm, out_hbm,
               src_v, dst_v, rows0_v, rows1_v, z_sh, sem0, sem1):
    c = lax.axis_index("c")
    s = lax.axis_index("s")
    wid = s * NUM_CORES + c

    pltpu.sync_copy(zeros_hbm.at[pl.ds(0, ROWS_PER_TILE)],
                    z_sh.at[pl.ds(s * ROWS_PER_TILE, ROWS_PER_TILE)])
    @pl.when(s == NUM_SUBCORES - 1)
    def _zero_tail():
        pltpu.sync_copy(
            zeros_hbm.at[pl.ds(0, ROWS_TAIL)],
            z_sh.at[pl.ds(NUM_SUBCORES * ROWS_PER_TILE, ROWS_TAIL)])
    plsc.subcore_barrier()

    for p in range(PHASES):
        pltpu.sync_copy(src_hbm.at[wid].at[p], src_v)
        pltpu.sync_copy(dst_hbm.at[wid].at[p], dst_v)
        def step(i, carry):
            pltpu.async_copy(m_hbm.at[src_v.at[i]], rows0_v, sem0).wait()
            pltpu.sync_copy(rows0_v, z_sh.at[dst_v.at[i]], add=True)
            return carry

        lax.fori_loop(0, PHASE_STEPS, step, 0)
    plsc.subcore_barrier()

    pltpu.sync_copy(
        z_sh.at[pl.ds(s * ROWS_PER_TILE, ROWS_PER_TILE)],
        out_hbm.at[c].at[pl.ds(s * ROWS_PER_TILE, ROWS_PER_TILE)])
    @pl.when(s == NUM_SUBCORES - 1)
    def _out_tail():
        pltpu.sync_copy(
            z_sh.at[pl.ds(NUM_SUBCORES * ROWS_PER_TILE, ROWS_TAIL)],
            out_hbm.at[c].at[pl.ds(NUM_SUBCORES * ROWS_PER_TILE, ROWS_TAIL)])


def kernel(x, edge_index, W1, b1, W2, b2, W3, b3, W4, b4):
    pad = PAD_EDGES_PER_TILE - EDGES_PER_TILE
    src = jnp.concatenate(
        [edge_index[0].astype(jnp.int32).reshape(NUM_TILES, EDGES_PER_TILE),
         jnp.zeros((NUM_TILES, pad), jnp.int32)],
        axis=1).reshape(NUM_TILES, PHASES, PHASE_STEPS, CHUNK)
    dst = jnp.concatenate(
        [edge_index[1].astype(jnp.int32).reshape(NUM_TILES, EDGES_PER_TILE),
         jnp.full((NUM_TILES, pad), N_NODES, jnp.int32)],
        axis=1).reshape(NUM_TILES, PHASES, PHASE_STEPS, CHUNK)
    m = _mlp1(x, W1, b1.reshape(1, DIM), W2, b2.reshape(1, DIM))
    zeros = jnp.zeros((max(ROWS_PER_TILE, ROWS_TAIL), DIM), jnp.float32)
    z_parts = _aggregate(m, src, dst, zeros)
    return _mlp2(z_parts[0], z_parts[1],
                 W3, b3.reshape(1, DIM), W4, b4.reshape(1, DIM))


# speedup vs baseline: 2.6886x; 2.6886x over previous
import functools

import jax
import jax.numpy as jnp
from jax import lax
from jax.experimental import pallas as pl
from jax.experimental.pallas import tpu as pltpu
from jax.experimental.pallas import tpu_sc as plsc

N_NODES = 10000
N_EDGES = 320000
DIM = 128

NUM_CORES = 2
NUM_SUBCORES = 16
NUM_TILES = NUM_CORES * NUM_SUBCORES

EDGES_PER_TILE = N_EDGES // NUM_TILES
CHUNK = 125
PHASES = 2
PHASE_STEPS = 40
STEPS = PHASES * PHASE_STEPS
ROWS_PER_TILE = 624
ROWS_TAIL = N_NODES - NUM_SUBCORES * ROWS_PER_TILE

_ROW_BLK = 2000


def _mlp1_body(x_ref, w1_ref, b1_ref, w2_ref, b2_ref, o_ref):
    h = jnp.maximum(
        jnp.dot(x_ref[...], w1_ref[...], preferred_element_type=jnp.float32)
        + b1_ref[...], 0.0)
    o_ref[...] = jnp.maximum(
        jnp.dot(h, w2_ref[...], preferred_element_type=jnp.float32)
        + b2_ref[...], 0.0)


def _mlp2_body(z0_ref, z1_ref, w3_ref, b3_ref, w4_ref, b4_ref, o_ref):
    z = z0_ref[...] + z1_ref[...]
    h = jnp.maximum(
        jnp.dot(z, w3_ref[...], preferred_element_type=jnp.float32)
        + b3_ref[...], 0.0)
    o_ref[...] = jnp.maximum(
        jnp.dot(h, w4_ref[...], preferred_element_type=jnp.float32)
        + b4_ref[...], 0.0)


_full = pl.BlockSpec((DIM, DIM), lambda i: (0, 0))
_bias = pl.BlockSpec((1, DIM), lambda i: (0, 0))
_rows = pl.BlockSpec((_ROW_BLK, DIM), lambda i: (i, 0))

_mlp1 = pl.pallas_call(
    _mlp1_body,
    grid=(N_NODES // _ROW_BLK,),
    in_specs=[_rows, _full, _bias, _full, _bias],
    out_specs=_rows,
    out_shape=jax.ShapeDtypeStruct((N_NODES, DIM), jnp.float32),
)

_mlp2 = pl.pallas_call(
    _mlp2_body,
    grid=(N_NODES // _ROW_BLK,),
    in_specs=[_rows, _rows, _full, _bias, _full, _bias],
    out_specs=_rows,
    out_shape=jax.ShapeDtypeStruct((N_NODES, DIM), jnp.float32),
)


@functools.partial(
    pl.kernel,
    out_type=jax.ShapeDtypeStruct((NUM_CORES, N_NODES, DIM), jnp.float32),
    mesh=plsc.VectorSubcoreMesh(core_axis_name="c", subcore_axis_name="s"),
    scratch_types=[
        pltpu.VMEM((PHASE_STEPS, CHUNK), jnp.int32),
        pltpu.VMEM((PHASE_STEPS, CHUNK), jnp.int32),
        pltpu.VMEM((CHUNK, DIM), jnp.float32),
        pltpu.VMEM((CHUNK, DIM), jnp.float32),
        pltpu.VMEM_SHARED((N_NODES, DIM), jnp.float32),
        pltpu.SemaphoreType.DMA,
        pltpu.SemaphoreType.DMA,
    ],
)
def _aggregate(m_hbm, src_hbm, dst_hbm, zeros_hbm, out_hbm,
               src_v, dst_v, rows0_v, rows1_v, z_sh, sem0, sem1):
    c = lax.axis_index("c")
    s = lax.axis_index("s")
    wid = s * NUM_CORES + c

    pltpu.sync_copy(zeros_hbm.at[pl.ds(0, ROWS_PER_TILE)],
                    z_sh.at[pl.ds(s * ROWS_PER_TILE, ROWS_PER_TILE)])
    @pl.when(s == NUM_SUBCORES - 1)
    def _zero_tail():
        pltpu.sync_copy(
            zeros_hbm.at[pl.ds(0, ROWS_TAIL)],
            z_sh.at[pl.ds(NUM_SUBCORES * ROWS_PER_TILE, ROWS_TAIL)])
    plsc.subcore_barrier()

    for p in range(PHASES):
        pltpu.sync_copy(src_hbm.at[wid].at[p], src_v)
        pltpu.sync_copy(dst_hbm.at[wid].at[p], dst_v)
        pltpu.async_copy(m_hbm.at[src_v.at[0]], rows0_v, sem0)

        def step(j, carry):
            i = 2 * j
            pltpu.make_async_copy(m_hbm.at[src_v.at[i]], rows0_v, sem0).wait()
            pltpu.async_copy(m_hbm.at[src_v.at[i + 1]], rows1_v, sem1)
            pltpu.sync_copy(rows0_v, z_sh.at[dst_v.at[i]], add=True)
            pltpu.make_async_copy(
                m_hbm.at[src_v.at[i + 1]], rows1_v, sem1).wait()

            @pl.when(j < PHASE_STEPS // 2 - 1)
            def _prefetch_next():
                pltpu.async_copy(m_hbm.at[src_v.at[i + 2]], rows0_v, sem0)

            pltpu.sync_copy(rows1_v, z_sh.at[dst_v.at[i + 1]], add=True)
            return carry

        lax.fori_loop(0, PHASE_STEPS // 2, step, 0)
    plsc.subcore_barrier()

    pltpu.sync_copy(
        z_sh.at[pl.ds(s * ROWS_PER_TILE, ROWS_PER_TILE)],
        out_hbm.at[c].at[pl.ds(s * ROWS_PER_TILE, ROWS_PER_TILE)])
    @pl.when(s == NUM_SUBCORES - 1)
    def _out_tail():
        pltpu.sync_copy(
            z_sh.at[pl.ds(NUM_SUBCORES * ROWS_PER_TILE, ROWS_TAIL)],
            out_hbm.at[c].at[pl.ds(NUM_SUBCORES * ROWS_PER_TILE, ROWS_TAIL)])


def kernel(x, edge_index, W1, b1, W2, b2, W3, b3, W4, b4):
    src = edge_index[0].astype(jnp.int32).reshape(
        NUM_TILES, PHASES, PHASE_STEPS, CHUNK)
    dst = edge_index[1].astype(jnp.int32).reshape(
        NUM_TILES, PHASES, PHASE_STEPS, CHUNK)
    m = _mlp1(x, W1, b1.reshape(1, DIM), W2, b2.reshape(1, DIM))
    zeros = jnp.zeros((ROWS_PER_TILE, DIM), jnp.float32)
    z_parts = _aggregate(m, src, dst, zeros)
    return _mlp2(z_parts[0], z_parts[1],
                 W3, b3.reshape(1, DIM), W4, b4.reshape(1, DIM))


# speedup vs baseline: 3.0313x; 1.1275x over previous
import functools

import jax
import jax.numpy as jnp
from jax import lax
from jax.experimental import pallas as pl
from jax.experimental.pallas import tpu as pltpu
from jax.experimental.pallas import tpu_sc as plsc

N_NODES = 10000
N_EDGES = 320000
DIM = 128

NUM_CORES = 2
NUM_SUBCORES = 16
NUM_TILES = NUM_CORES * NUM_SUBCORES

EDGES_PER_TILE = N_EDGES // NUM_TILES
CHUNK = 125
PHASES = 2
PHASE_STEPS = 40
STEPS = PHASES * PHASE_STEPS
ROWS_PER_TILE = 624
ROWS_TAIL = N_NODES - NUM_SUBCORES * ROWS_PER_TILE

_ROW_BLK = 2000


def _mlp1_body(x_ref, w1_ref, b1_ref, w2_ref, b2_ref, o_ref):
    h = jnp.maximum(
        jnp.dot(x_ref[...], w1_ref[...], preferred_element_type=jnp.float32)
        + b1_ref[...], 0.0)
    o_ref[...] = jnp.maximum(
        jnp.dot(h, w2_ref[...], preferred_element_type=jnp.float32)
        + b2_ref[...], 0.0)


def _mlp2_body(z0_ref, z1_ref, w3_ref, b3_ref, w4_ref, b4_ref, o_ref):
    z = z0_ref[...] + z1_ref[...]
    h = jnp.maximum(
        jnp.dot(z, w3_ref[...], preferred_element_type=jnp.float32)
        + b3_ref[...], 0.0)
    o_ref[...] = jnp.maximum(
        jnp.dot(h, w4_ref[...], preferred_element_type=jnp.float32)
        + b4_ref[...], 0.0)


_full = pl.BlockSpec((DIM, DIM), lambda i: (0, 0))
_bias = pl.BlockSpec((1, DIM), lambda i: (0, 0))
_rows = pl.BlockSpec((_ROW_BLK, DIM), lambda i: (i, 0))

_mlp1 = pl.pallas_call(
    _mlp1_body,
    grid=(N_NODES // _ROW_BLK,),
    in_specs=[_rows, _full, _bias, _full, _bias],
    out_specs=_rows,
    out_shape=jax.ShapeDtypeStruct((N_NODES, DIM), jnp.float32),
)

_mlp2 = pl.pallas_call(
    _mlp2_body,
    grid=(N_NODES // _ROW_BLK,),
    in_specs=[_rows, _rows, _full, _bias, _full, _bias],
    out_specs=_rows,
    out_shape=jax.ShapeDtypeStruct((N_NODES, DIM), jnp.float32),
)


@functools.partial(
    pl.kernel,
    out_type=jax.ShapeDtypeStruct((NUM_CORES, N_NODES, DIM), jnp.float32),
    mesh=plsc.VectorSubcoreMesh(core_axis_name="c", subcore_axis_name="s"),
    scratch_types=[
        pltpu.VMEM((PHASE_STEPS, CHUNK), jnp.int32),
        pltpu.VMEM((PHASE_STEPS, CHUNK), jnp.int32),
        pltpu.VMEM((CHUNK, DIM), jnp.float32),
        pltpu.VMEM((CHUNK, DIM), jnp.float32),
        pltpu.VMEM_SHARED((N_NODES, DIM), jnp.float32),
        pltpu.SemaphoreType.DMA,
        pltpu.SemaphoreType.DMA,
    ],
)
def _aggregate(m_hbm, src_hbm, dst_hbm, zeros_hbm, out_hbm,
               src_v, dst_v, rows0_v, rows1_v, z_sh, sem0, sem1):
    c = lax.axis_index("c")
    s = lax.axis_index("s")
    wid = s * NUM_CORES + c

    pltpu.sync_copy(zeros_hbm.at[pl.ds(0, ROWS_PER_TILE)],
                    z_sh.at[pl.ds(s * ROWS_PER_TILE, ROWS_PER_TILE)])
    @pl.when(s == NUM_SUBCORES - 1)
    def _zero_tail():
        pltpu.sync_copy(
            zeros_hbm.at[pl.ds(0, ROWS_TAIL)],
            z_sh.at[pl.ds(NUM_SUBCORES * ROWS_PER_TILE, ROWS_TAIL)])
    plsc.subcore_barrier()

    for p in range(PHASES):
        pltpu.sync_copy(src_hbm.at[wid].at[p], src_v)
        pltpu.sync_copy(dst_hbm.at[wid].at[p], dst_v)
        pltpu.async_copy(m_hbm.at[src_v.at[0]], rows0_v, sem0)
        pltpu.async_copy(m_hbm.at[src_v.at[1]], rows1_v, sem1)

        def step(j, carry):
            i = 2 * j
            pltpu.make_async_copy(m_hbm.at[src_v.at[i]], rows0_v, sem0).wait()
            pltpu.sync_copy(rows0_v, z_sh.at[dst_v.at[i]], add=True)

            @pl.when(j < PHASE_STEPS // 2 - 1)
            def _refill0():
                pltpu.async_copy(m_hbm.at[src_v.at[i + 2]], rows0_v, sem0)

            pltpu.make_async_copy(
                m_hbm.at[src_v.at[i + 1]], rows1_v, sem1).wait()
            pltpu.sync_copy(rows1_v, z_sh.at[dst_v.at[i + 1]], add=True)

            @pl.when(j < PHASE_STEPS // 2 - 1)
            def _refill1():
                pltpu.async_copy(m_hbm.at[src_v.at[i + 3]], rows1_v, sem1)

            return carry

        lax.fori_loop(0, PHASE_STEPS // 2, step, 0)
    plsc.subcore_barrier()

    pltpu.sync_copy(
        z_sh.at[pl.ds(s * ROWS_PER_TILE, ROWS_PER_TILE)],
        out_hbm.at[c].at[pl.ds(s * ROWS_PER_TILE, ROWS_PER_TILE)])
    @pl.when(s == NUM_SUBCORES - 1)
    def _out_tail():
        pltpu.sync_copy(
            z_sh.at[pl.ds(NUM_SUBCORES * ROWS_PER_TILE, ROWS_TAIL)],
            out_hbm.at[c].at[pl.ds(NUM_SUBCORES * ROWS_PER_TILE, ROWS_TAIL)])


def kernel(x, edge_index, W1, b1, W2, b2, W3, b3, W4, b4):
    src = edge_index[0].astype(jnp.int32).reshape(
        NUM_TILES, PHASES, PHASE_STEPS, CHUNK)
    dst = edge_index[1].astype(jnp.int32).reshape(
        NUM_TILES, PHASES, PHASE_STEPS, CHUNK)
    m = _mlp1(x, W1, b1.reshape(1, DIM), W2, b2.reshape(1, DIM))
    zeros = jnp.zeros((ROWS_PER_TILE, DIM), jnp.float32)
    z_parts = _aggregate(m, src, dst, zeros)
    return _mlp2(z_parts[0], z_parts[1],
                 W3, b3.reshape(1, DIM), W4, b4.reshape(1, DIM))


# speedup vs baseline: 3.1325x; 1.0334x over previous
import functools

import jax
import jax.numpy as jnp
from jax import lax
from jax.experimental import pallas as pl
from jax.experimental.pallas import tpu as pltpu
from jax.experimental.pallas import tpu_sc as plsc

N_NODES = 10000
N_EDGES = 320000
DIM = 128

NUM_CORES = 2
NUM_SUBCORES = 16
NUM_TILES = NUM_CORES * NUM_SUBCORES

EDGES_PER_TILE = N_EDGES // NUM_TILES
CHUNK = 50
PHASES = 4
PHASE_STEPS = 50
STEPS = PHASES * PHASE_STEPS
NBUF = 5
ROWS_PER_TILE = 624
ROWS_TAIL = N_NODES - NUM_SUBCORES * ROWS_PER_TILE

_ROW_BLK = 2000


def _mlp1_body(x_ref, w1_ref, b1_ref, w2_ref, b2_ref, o_ref):
    h = jnp.maximum(
        jnp.dot(x_ref[...], w1_ref[...], preferred_element_type=jnp.float32)
        + b1_ref[...], 0.0)
    o_ref[...] = jnp.maximum(
        jnp.dot(h, w2_ref[...], preferred_element_type=jnp.float32)
        + b2_ref[...], 0.0)


def _mlp2_body(z0_ref, z1_ref, w3_ref, b3_ref, w4_ref, b4_ref, o_ref):
    z = z0_ref[...] + z1_ref[...]
    h = jnp.maximum(
        jnp.dot(z, w3_ref[...], preferred_element_type=jnp.float32)
        + b3_ref[...], 0.0)
    o_ref[...] = jnp.maximum(
        jnp.dot(h, w4_ref[...], preferred_element_type=jnp.float32)
        + b4_ref[...], 0.0)


_full = pl.BlockSpec((DIM, DIM), lambda i: (0, 0))
_bias = pl.BlockSpec((1, DIM), lambda i: (0, 0))
_rows = pl.BlockSpec((_ROW_BLK, DIM), lambda i: (i, 0))

_mlp1 = pl.pallas_call(
    _mlp1_body,
    grid=(N_NODES // _ROW_BLK,),
    in_specs=[_rows, _full, _bias, _full, _bias],
    out_specs=_rows,
    out_shape=jax.ShapeDtypeStruct((N_NODES, DIM), jnp.float32),
)

_mlp2 = pl.pallas_call(
    _mlp2_body,
    grid=(N_NODES // _ROW_BLK,),
    in_specs=[_rows, _rows, _full, _bias, _full, _bias],
    out_specs=_rows,
    out_shape=jax.ShapeDtypeStruct((N_NODES, DIM), jnp.float32),
)


@functools.partial(
    pl.kernel,
    out_type=jax.ShapeDtypeStruct((NUM_CORES, N_NODES, DIM), jnp.float32),
    mesh=plsc.VectorSubcoreMesh(core_axis_name="c", subcore_axis_name="s"),
    scratch_types=[
        pltpu.VMEM((PHASE_STEPS, CHUNK), jnp.int32),
        pltpu.VMEM((PHASE_STEPS, CHUNK), jnp.int32),
        [pltpu.VMEM((CHUNK, DIM), jnp.float32)] * NBUF,
        pltpu.VMEM_SHARED((N_NODES, DIM), jnp.float32),
        [pltpu.SemaphoreType.DMA] * NBUF,
    ],
)
def _aggregate(m_hbm, src_hbm, dst_hbm, zeros_hbm, out_hbm,
               src_v, dst_v, rows_v, z_sh, sems):
    c = lax.axis_index("c")
    s = lax.axis_index("s")
    wid = s * NUM_CORES + c

    pltpu.sync_copy(zeros_hbm.at[pl.ds(0, ROWS_PER_TILE)],
                    z_sh.at[pl.ds(s * ROWS_PER_TILE, ROWS_PER_TILE)])
    @pl.when(s == NUM_SUBCORES - 1)
    def _zero_tail():
        pltpu.sync_copy(
            zeros_hbm.at[pl.ds(0, ROWS_TAIL)],
            z_sh.at[pl.ds(NUM_SUBCORES * ROWS_PER_TILE, ROWS_TAIL)])
    plsc.subcore_barrier()

    for p in range(PHASES):
        pltpu.sync_copy(src_hbm.at[wid].at[p], src_v)
        pltpu.sync_copy(dst_hbm.at[wid].at[p], dst_v)
        for b in range(NBUF):
            pltpu.async_copy(m_hbm.at[src_v.at[b]], rows_v[b], sems[b])

        def step(j, carry):
            base = NBUF * j
            for b in range(NBUF):
                i = base + b
                pltpu.make_async_copy(
                    m_hbm.at[src_v.at[i]], rows_v[b], sems[b]).wait()
                pltpu.sync_copy(rows_v[b], z_sh.at[dst_v.at[i]], add=True)

                @pl.when(i + NBUF < PHASE_STEPS)
                def _refill():
                    pltpu.async_copy(
                        m_hbm.at[src_v.at[i + NBUF]], rows_v[b], sems[b])
            return carry

        lax.fori_loop(0, PHASE_STEPS // NBUF, step, 0)
    plsc.subcore_barrier()

    pltpu.sync_copy(
        z_sh.at[pl.ds(s * ROWS_PER_TILE, ROWS_PER_TILE)],
        out_hbm.at[c].at[pl.ds(s * ROWS_PER_TILE, ROWS_PER_TILE)])
    @pl.when(s == NUM_SUBCORES - 1)
    def _out_tail():
        pltpu.sync_copy(
            z_sh.at[pl.ds(NUM_SUBCORES * ROWS_PER_TILE, ROWS_TAIL)],
            out_hbm.at[c].at[pl.ds(NUM_SUBCORES * ROWS_PER_TILE, ROWS_TAIL)])


def kernel(x, edge_index, W1, b1, W2, b2, W3, b3, W4, b4):
    src = edge_index[0].astype(jnp.int32).reshape(
        NUM_TILES, PHASES, PHASE_STEPS, CHUNK)
    dst = edge_index[1].astype(jnp.int32).reshape(
        NUM_TILES, PHASES, PHASE_STEPS, CHUNK)
    m = _mlp1(x, W1, b1.reshape(1, DIM), W2, b2.reshape(1, DIM))
    zeros = jnp.zeros((ROWS_PER_TILE, DIM), jnp.float32)
    z_parts = _aggregate(m, src, dst, zeros)
    return _mlp2(z_parts[0], z_parts[1],
                 W3, b3.reshape(1, DIM), W4, b4.reshape(1, DIM))
